# baseline (speedup 1.0000x reference)
import functools

import jax
import jax.numpy as jnp
from jax.experimental import pallas as pl

B = 2048
DIM = 512
PROJ = 768
TOPK = 64
BM = 256
NB = B // BM
FILL = 1.0e9
STEP = 1.0e4
VAL_ITERS = 22
IDX_ITERS = 12


def _dot_nt(a, b):
    return jax.lax.dot_general(a, b, (((1,), (1,)), ((), ())),
                               preferred_element_type=jnp.float32)


def _row_ids(i):
    col = jax.lax.broadcasted_iota(jnp.int32, (BM, B), 1).astype(jnp.float32)
    row = (jax.lax.broadcasted_iota(jnp.int32, (BM, B), 0) + i * BM).astype(
        jnp.float32)
    return col, row


def _sim_kernel(xi_ref, xj_ref, s_ref):
    xi = xi_ref[...]
    xj = xj_ref[...]
    ni = jnp.clip(jnp.sqrt(jnp.sum(xi * xi, axis=1, keepdims=True)), 1e-12,
                  None)
    nj = jnp.clip(jnp.sqrt(jnp.sum(xj * xj, axis=1, keepdims=True)), 1e-12,
                  None)
    s = _dot_nt(xi / ni, xj / nj)
    s_ref[...] = jnp.clip(s, -1.0 + 1e-8, 1.0 - 1e-8)


def _similarity(x):
    return pl.pallas_call(
        _sim_kernel,
        grid=(NB, NB),
        in_specs=[
            pl.BlockSpec((BM, DIM), lambda i, j: (i, 0)),
            pl.BlockSpec((BM, DIM), lambda i, j: (j, 0)),
        ],
        out_specs=pl.BlockSpec((BM, BM), lambda i, j: (i, j)),
        out_shape=jax.ShapeDtypeStruct((B, B), jnp.float32),
    )(x, x)


def _masked_pair(s, base, col):
    return base * s - (1.0 - base) * (FILL + STEP * col)


def _thr_kernel(s_ref, labr_ref, labc_ref, ta_ref, te_ref):
    i = pl.program_id(0)
    k = float(TOPK)
    s = s_ref[...]
    same = labc_ref[...] == labr_ref[...]
    col, row = _row_ids(i)
    noneye = col != row
    in_a = jnp.logical_and(same, noneye)
    in_e = jnp.logical_and(jnp.logical_not(same), noneye)

    def rowcount(cond):
        return jnp.sum(jnp.where(cond, 1.0, 0.0), axis=1, keepdims=True)

    n_a = rowcount(in_a)
    n_e = rowcount(in_e)

    def vstep(c):
        la, ha, le, he = c
        ma = (la + ha) * 0.5
        me = (le + he) * 0.5
        ca = rowcount(jnp.logical_and(in_a, s >= ma))
        ce = rowcount(jnp.logical_and(in_e, s >= me))
        oa = ca >= k
        oe = ce >= k
        return (jnp.where(oa, ma, la), jnp.where(oa, ha, ma),
                jnp.where(oe, me, le), jnp.where(oe, he, me))

    def istep(c):
        la, ha, le, he = c
        ma = jnp.floor((la + ha) * 0.5)
        me = jnp.floor((le + he) * 0.5)
        ca = n_a + rowcount(jnp.logical_and(jnp.logical_not(in_a), col <= ma))
        ce = n_e + rowcount(jnp.logical_and(jnp.logical_not(in_e), col <= me))
        oa = ca >= k
        oe = ce >= k
        return (jnp.where(oa, la, ma), jnp.where(oa, ma, ha),
                jnp.where(oe, le, me), jnp.where(oe, me, he))

    def body_both(_, c):
        vc, ic = c
        return vstep(vc), istep(ic)

    def body_val(_, c):
        vc, ic = c
        return vstep(vc), ic

    lo0 = jnp.full((BM, 1), -2.0, jnp.float32)
    hi0 = jnp.full((BM, 1), 2.0, jnp.float32)
    ilo0 = jnp.full((BM, 1), -1.0, jnp.float32)
    ihi0 = jnp.full((BM, 1), float(B), jnp.float32)
    c0 = ((lo0, hi0, lo0, hi0), (ilo0, ihi0, ilo0, ihi0))
    c1 = jax.lax.fori_loop(0, IDX_ITERS, body_both, c0)
    (la, _, le, _), (_, iha, _, ihe) = jax.lax.fori_loop(
        0, VAL_ITERS - IDX_ITERS, body_val, c1)
    ta_ref[...] = jnp.where(n_a >= k, la, -(FILL + STEP * iha))
    te_ref[...] = jnp.where(n_e >= k, le, -(FILL + STEP * ihe))


def _thresholds(s, lab_row, lab_col):
    return pl.pallas_call(
        _thr_kernel,
        grid=(NB,),
        in_specs=[
            pl.BlockSpec((BM, B), lambda i: (i, 0)),
            pl.BlockSpec((1, B), lambda i: (0, 0)),
            pl.BlockSpec((BM, 1), lambda i: (i, 0)),
        ],
        out_specs=[
            pl.BlockSpec((BM, 1), lambda i: (i, 0)),
            pl.BlockSpec((BM, 1), lambda i: (i, 0)),
        ],
        out_shape=[
            jax.ShapeDtypeStruct((B, 1), jnp.float32),
            jax.ShapeDtypeStruct((B, 1), jnp.float32),
        ],
    )(s, lab_row, lab_col)


def _adj_kernel(a_coef, b_coef, s_ref, labr_ref, labc_ref, tar_ref, tac_ref,
                ter_ref, tec_ref, a_ref, dinv_ref):
    i = pl.program_id(0)
    s = s_ref[...]
    same = (labc_ref[...] == labr_ref[...]).astype(jnp.float32)
    col, row = _row_ids(i)
    noneye = (col != row).astype(jnp.float32)
    sa = same * noneye
    se = (1.0 - same) * noneye

    ma = _masked_pair(s, sa, col) >= tac_ref[...]
    ma_t = _masked_pair(s, sa, row) >= tar_ref[...]
    me = _masked_pair(s, se, col) >= tec_ref[...]
    me_t = _masked_pair(s, se, row) >= ter_ref[...]
    m_intra = jnp.logical_or(ma, ma_t).astype(jnp.float32)
    m_inter = jnp.logical_or(me, me_t).astype(jnp.float32)

    s_pos = jnp.maximum(s, 0.0)
    eye = 1.0 - noneye
    a = a_coef * s_pos * m_intra + b_coef * s_pos * m_inter + 1e-6 * eye
    a_ref[...] = a
    d = jnp.clip(jnp.sum(a, axis=1, keepdims=True), 1e-8, None)
    dinv_ref[...] = jax.lax.rsqrt(d)


def _adjacency(s, lab_row, lab_col, thr_a, thr_e, a_coef, b_coef):
    thr_a_row = thr_a.reshape(1, B)
    thr_e_row = thr_e.reshape(1, B)
    return pl.pallas_call(
        functools.partial(_adj_kernel, a_coef, b_coef),
        grid=(NB,),
        in_specs=[
            pl.BlockSpec((BM, B), lambda i: (i, 0)),
            pl.BlockSpec((1, B), lambda i: (0, 0)),
            pl.BlockSpec((BM, 1), lambda i: (i, 0)),
            pl.BlockSpec((1, B), lambda i: (0, 0)),
            pl.BlockSpec((BM, 1), lambda i: (i, 0)),
            pl.BlockSpec((1, B), lambda i: (0, 0)),
            pl.BlockSpec((BM, 1), lambda i: (i, 0)),
        ],
        out_specs=[
            pl.BlockSpec((BM, B), lambda i: (i, 0)),
            pl.BlockSpec((BM, 1), lambda i: (i, 0)),
        ],
        out_shape=[
            jax.ShapeDtypeStruct((B, B), jnp.float32),
            jax.ShapeDtypeStruct((B, 1), jnp.float32),
        ],
    )(s, lab_row, lab_col, thr_a_row, thr_a, thr_e_row, thr_e)


def _diff_kernel(ai_ref, aj_ref, dr_ref, di_ref, dj_ref, k_ref):
    dinv2 = dr_ref[...] * dr_ref[...]
    kk = _dot_nt(ai_ref[...] * dinv2, aj_ref[...])
    k_ref[...] = kk * di_ref[...] * dj_ref[...]


def _diffusion(a, dinv_col):
    dinv_row = dinv_col.reshape(1, B)
    return pl.pallas_call(
        _diff_kernel,
        grid=(NB, NB),
        in_specs=[
            pl.BlockSpec((BM, B), lambda i, j: (i, 0)),
            pl.BlockSpec((BM, B), lambda i, j: (j, 0)),
            pl.BlockSpec((1, B), lambda i, j: (0, 0)),
            pl.BlockSpec((BM, 1), lambda i, j: (i, 0)),
            pl.BlockSpec((1, BM), lambda i, j: (0, j)),
        ],
        out_specs=pl.BlockSpec((BM, BM), lambda i, j: (i, j)),
        out_shape=jax.ShapeDtypeStruct((B, B), jnp.float32),
    )(a, a, dinv_row, dinv_col, dinv_row)


def _mm_nt_kernel(x_ref, w_ref, y_ref):
    y_ref[...] = _dot_nt(x_ref[...], w_ref[...])


def _mm_nt(x, w):
    n = w.shape[0]
    return pl.pallas_call(
        _mm_nt_kernel,
        grid=(NB,),
        in_specs=[
            pl.BlockSpec((BM, x.shape[1]), lambda i: (i, 0)),
            pl.BlockSpec(w.shape, lambda i: (0, 0)),
        ],
        out_specs=pl.BlockSpec((BM, n), lambda i: (i, 0)),
        out_shape=jax.ShapeDtypeStruct((B, n), jnp.float32),
    )(x, w)


def _applyA_res_kernel(a_ref, y_ref, dc_ref, di_ref, res_ref, z_ref):
    yd = y_ref[...] * dc_ref[...]
    z = jnp.dot(a_ref[...], yd, preferred_element_type=jnp.float32)
    z_ref[...] = z * di_ref[...] + res_ref[...]


def _applyA_kernel(a_ref, y_ref, dc_ref, di_ref, z_ref):
    yd = y_ref[...] * dc_ref[...]
    z = jnp.dot(a_ref[...], yd, preferred_element_type=jnp.float32)
    z_ref[...] = z * di_ref[...]


def _applyA(a, y, dinv_col, res=None):
    specs = [
        pl.BlockSpec((BM, B), lambda i: (i, 0)),
        pl.BlockSpec((B, DIM), lambda i: (0, 0)),
        pl.BlockSpec((B, 1), lambda i: (0, 0)),
        pl.BlockSpec((BM, 1), lambda i: (i, 0)),
    ]
    args = [a, y, dinv_col, dinv_col]
    fn = _applyA_kernel
    if res is not None:
        specs.append(pl.BlockSpec((BM, DIM), lambda i: (i, 0)))
        args.append(res)
        fn = _applyA_res_kernel
    return pl.pallas_call(
        fn,
        grid=(NB,),
        in_specs=specs,
        out_specs=pl.BlockSpec((BM, DIM), lambda i: (i, 0)),
        out_shape=jax.ShapeDtypeStruct((B, DIM), jnp.float32),
    )(*args)


def _bn_kernel(x_ref, g_ref, b_ref, o_ref):
    x = x_ref[...]
    mu = jnp.mean(x, axis=0, keepdims=True)
    xc = x - mu
    var = jnp.mean(xc * xc, axis=0, keepdims=True)
    bn = g_ref[...] * xc / jnp.sqrt(var + 1e-5) + b_ref[...]
    o_ref[...] = jnp.maximum(bn, 0.0)


def _bn_relu(x, g, b):
    return pl.pallas_call(
        _bn_kernel,
        grid=(1,),
        in_specs=[
            pl.BlockSpec((B, DIM), lambda i: (0, 0)),
            pl.BlockSpec((1, DIM), lambda i: (0, 0)),
            pl.BlockSpec((1, DIM), lambda i: (0, 0)),
        ],
        out_specs=pl.BlockSpec((B, DIM), lambda i: (0, 0)),
        out_shape=jax.ShapeDtypeStruct((B, DIM), jnp.float32),
    )(x, g.reshape(1, DIM), b.reshape(1, DIM))


def _cnt_kernel(labr_ref, labc_ref, c_ref):
    same = (labc_ref[...] == labr_ref[...]).astype(jnp.float32)
    c_ref[...] = jnp.sum(same, axis=1, keepdims=True)


def _counts(lab_row, lab_col):
    return pl.pallas_call(
        _cnt_kernel,
        grid=(NB,),
        in_specs=[
            pl.BlockSpec((1, B), lambda i: (0, 0)),
            pl.BlockSpec((BM, 1), lambda i: (i, 0)),
        ],
        out_specs=pl.BlockSpec((BM, 1), lambda i: (i, 0)),
        out_shape=jax.ShapeDtypeStruct((B, 1), jnp.float32),
    )(lab_row, lab_col)


def _idea_dinv(cnt):
    d = 0.98 * (cnt - 1.0) + 1.0 + 0.03 * (float(B) - cnt)
    return jax.lax.rsqrt(jnp.clip(d, 1e-8, None))


def _kloss_kernel(k0_ref, k1_ref, labr_ref, labc_ref, cr_ref, cc_ref,
                  p0_ref, p1_ref):
    i = pl.program_id(0)
    k0 = k0_ref[...]
    k1 = k1_ref[...]
    same = (labc_ref[...] == labr_ref[...]).astype(jnp.float32)
    col, row = _row_ids(i)
    eye = (col == row).astype(jnp.float32)
    v = jnp.maximum(same * 0.98 + (1.0 - same) * 0.03, eye)
    k_idea = v * _idea_dinv(cc_ref[...]) * _idea_dinv(cr_ref[...])
    d0 = k0 - k1
    d1 = k1 - k_idea
    p0_ref[...] = jnp.full((BM, 1), jnp.sum(d0 * d0), jnp.float32)
    p1_ref[...] = jnp.full((BM, 1), jnp.sum(d1 * d1), jnp.float32)


def _kloss(k0, k1, lab_row, lab_col, cnt):
    cnt_row = cnt.reshape(1, B)
    return pl.pallas_call(
        _kloss_kernel,
        grid=(NB,),
        in_specs=[
            pl.BlockSpec((BM, B), lambda i: (i, 0)),
            pl.BlockSpec((BM, B), lambda i: (i, 0)),
            pl.BlockSpec((1, B), lambda i: (0, 0)),
            pl.BlockSpec((BM, 1), lambda i: (i, 0)),
            pl.BlockSpec((1, B), lambda i: (0, 0)),
            pl.BlockSpec((BM, 1), lambda i: (i, 0)),
        ],
        out_specs=[
            pl.BlockSpec((BM, 1), lambda i: (i, 0)),
            pl.BlockSpec((BM, 1), lambda i: (i, 0)),
        ],
        out_shape=[
            jax.ShapeDtypeStruct((B, 1), jnp.float32),
            jax.ShapeDtypeStruct((B, 1), jnp.float32),
        ],
    )(k0, k1, lab_row, lab_col, cnt_row, cnt)


def _zloss_kernel(z0_ref, z1_ref, wp_ref, p_ref):
    wp = wp_ref[...]
    p0 = _dot_nt(z0_ref[...], wp)
    p1 = _dot_nt(z1_ref[...], wp)
    n0 = jnp.clip(jnp.sqrt(jnp.sum(p0 * p0, axis=1, keepdims=True)), 1e-12,
                  None)
    n1 = jnp.clip(jnp.sqrt(jnp.sum(p1 * p1, axis=1, keepdims=True)), 1e-12,
                  None)
    d = p0 / n0 - p1 / n1
    p_ref[...] = jnp.full((BM, 1), jnp.sum(d * d), jnp.float32)


def _zloss(z0, z1, wp):
    return pl.pallas_call(
        _zloss_kernel,
        grid=(NB,),
        in_specs=[
            pl.BlockSpec((BM, DIM), lambda i: (i, 0)),
            pl.BlockSpec((BM, DIM), lambda i: (i, 0)),
            pl.BlockSpec((PROJ, DIM), lambda i: (0, 0)),
        ],
        out_specs=pl.BlockSpec((BM, 1), lambda i: (i, 0)),
        out_shape=jax.ShapeDtypeStruct((B, 1), jnp.float32),
    )(z0, z1, wp)


def _psum(p):
    return jnp.sum(p[::BM, 0])


def _layer(x, lab_row, lab_col, w1, w2, g, b, a_coef, b_coef):
    s = _similarity(x)
    thr_a, thr_e = _thresholds(s, lab_row, lab_col)
    a, dinv = _adjacency(s, lab_row, lab_col, thr_a, thr_e, a_coef, b_coef)
    k = _diffusion(a, dinv)
    y1 = _mm_nt(x, w1)
    z1 = _applyA(a, y1, dinv)
    z1 = _bn_relu(z1, g, b)
    y2 = _mm_nt(z1, w2)
    z = _applyA(a, y2, dinv, res=x)
    return k, z


def kernel(feats_512_list, labels, W1_0, W2_0, g_0, b_0, W1_1, W2_1, g_1,
           b_1, Wp):
    labf = labels.astype(jnp.float32)
    lab_row = labf.reshape(1, B)
    lab_col = labf.reshape(B, 1)
    k0, z0 = _layer(feats_512_list[0], lab_row, lab_col, W1_0, W2_0, g_0,
                    b_0, 1.0, 1.0)
    k1, z1 = _layer(feats_512_list[1], lab_row, lab_col, W1_1, W2_1, g_1,
                    b_1, 1.2, 0.8)
    cnt = _counts(lab_row, lab_col)
    pk0, pk1 = _kloss(k0, k1, lab_row, lab_col, cnt)
    loss_align_K = _psum(pk0) / (float(B) * float(B))
    loss_idea = _psum(pk1) / (float(B) * float(B))
    pz = _zloss(z0, z1, Wp)
    loss_align_Z = _psum(pz) / (float(B) * float(PROJ))
    loss_pga = 0.5 * loss_align_K + 1.0 * loss_align_Z + 1.0 * loss_idea
    return (z0, z1, loss_align_K, loss_align_Z, loss_idea, loss_pga)


# speedup vs baseline: 1.0938x; 1.0938x over previous
import functools

import jax
import jax.numpy as jnp
from jax.experimental import pallas as pl

B = 2048
DIM = 512
PROJ = 768
TOPK = 64
BM = 256
NB = B // BM
FILL = 1.0e9
STEP = 1.0e4
VAL_ITERS = 22
IDX_ITERS = 12


def _dot_nt(a, b):
    return jax.lax.dot_general(a, b, (((1,), (1,)), ((), ())),
                               preferred_element_type=jnp.float32)


def _row_ids(i):
    col = jax.lax.broadcasted_iota(jnp.int32, (BM, B), 1).astype(jnp.float32)
    row = (jax.lax.broadcasted_iota(jnp.int32, (BM, B), 0) + i * BM).astype(
        jnp.float32)
    return col, row


def _sim_kernel(xi_ref, xj_ref, s_ref):
    xi = xi_ref[...]
    xj = xj_ref[...]
    ni = jnp.clip(jnp.sqrt(jnp.sum(xi * xi, axis=1, keepdims=True)), 1e-12,
                  None)
    nj = jnp.clip(jnp.sqrt(jnp.sum(xj * xj, axis=1, keepdims=True)), 1e-12,
                  None)
    s = _dot_nt(xi / ni, xj / nj)
    s_ref[...] = jnp.clip(s, -1.0 + 1e-8, 1.0 - 1e-8)


def _similarity(x):
    return pl.pallas_call(
        _sim_kernel,
        grid=(NB, NB),
        in_specs=[
            pl.BlockSpec((BM, DIM), lambda i, j: (i, 0)),
            pl.BlockSpec((BM, DIM), lambda i, j: (j, 0)),
        ],
        out_specs=pl.BlockSpec((BM, BM), lambda i, j: (i, j)),
        out_shape=jax.ShapeDtypeStruct((B, B), jnp.float32),
    )(x, x)


def _masked_pair(s, base, col):
    return base * s - (1.0 - base) * (FILL + STEP * col)


def _kth_threshold(masked):
    k = float(TOPK)
    n_real = jnp.sum((masked >= -2.0).astype(jnp.float32), axis=1,
                     keepdims=True)

    def vbody(_, c):
        lo, hi = c
        mid = (lo + hi) * 0.5
        cnt = jnp.sum((masked >= mid).astype(jnp.float32), axis=1,
                      keepdims=True)
        ok = cnt >= k
        return jnp.where(ok, mid, lo), jnp.where(ok, hi, mid)

    lo0 = jnp.full((BM, 1), -2.0, jnp.float32)
    hi0 = jnp.full((BM, 1), 2.0, jnp.float32)
    thr_val, _ = jax.lax.fori_loop(0, VAL_ITERS, vbody, (lo0, hi0))

    def ibody(_, c):
        lo, hi = c
        mid = jnp.floor((lo + hi) * 0.5)
        cnt = jnp.sum((masked >= -(FILL + STEP * mid)).astype(jnp.float32),
                      axis=1, keepdims=True)
        ok = cnt >= k
        return jnp.where(ok, lo, mid), jnp.where(ok, mid, hi)

    ilo0 = jnp.full((BM, 1), -1.0, jnp.float32)
    ihi0 = jnp.full((BM, 1), float(B), jnp.float32)
    _, t_star = jax.lax.fori_loop(0, IDX_ITERS, ibody, (ilo0, ihi0))
    thr_idx = -(FILL + STEP * t_star)
    return jnp.where(n_real >= k, thr_val, thr_idx)


def _thr_kernel(s_ref, labr_ref, labc_ref, ta_ref, te_ref):
    i = pl.program_id(0)
    s = s_ref[...]
    same = (labc_ref[...] == labr_ref[...]).astype(jnp.float32)
    col, row = _row_ids(i)
    noneye = (col != row).astype(jnp.float32)
    ta_ref[...] = _kth_threshold(_masked_pair(s, same * noneye, col))
    te_ref[...] = _kth_threshold(_masked_pair(s, (1.0 - same) * noneye, col))


def _thresholds(s, lab_row, lab_col):
    return pl.pallas_call(
        _thr_kernel,
        grid=(NB,),
        in_specs=[
            pl.BlockSpec((BM, B), lambda i: (i, 0)),
            pl.BlockSpec((1, B), lambda i: (0, 0)),
            pl.BlockSpec((BM, 1), lambda i: (i, 0)),
        ],
        out_specs=[
            pl.BlockSpec((BM, 1), lambda i: (i, 0)),
            pl.BlockSpec((BM, 1), lambda i: (i, 0)),
        ],
        out_shape=[
            jax.ShapeDtypeStruct((B, 1), jnp.float32),
            jax.ShapeDtypeStruct((B, 1), jnp.float32),
        ],
    )(s, lab_row, lab_col)


def _adj_kernel(a_coef, b_coef, s_ref, labr_ref, labc_ref, tar_ref, tac_ref,
                ter_ref, tec_ref, a_ref, dinv_ref):
    i = pl.program_id(0)
    s = s_ref[...]
    same = (labc_ref[...] == labr_ref[...]).astype(jnp.float32)
    col, row = _row_ids(i)
    noneye = (col != row).astype(jnp.float32)
    sa = same * noneye
    se = (1.0 - same) * noneye

    ma = _masked_pair(s, sa, col) >= tac_ref[...]
    ma_t = _masked_pair(s, sa, row) >= tar_ref[...]
    me = _masked_pair(s, se, col) >= tec_ref[...]
    me_t = _masked_pair(s, se, row) >= ter_ref[...]
    m_intra = jnp.logical_or(ma, ma_t).astype(jnp.float32)
    m_inter = jnp.logical_or(me, me_t).astype(jnp.float32)

    s_pos = jnp.maximum(s, 0.0)
    eye = 1.0 - noneye
    a = a_coef * s_pos * m_intra + b_coef * s_pos * m_inter + 1e-6 * eye
    a_ref[...] = a
    d = jnp.clip(jnp.sum(a, axis=1, keepdims=True), 1e-8, None)
    dinv_ref[...] = jax.lax.rsqrt(d)


def _adjacency(s, lab_row, lab_col, thr_a, thr_e, a_coef, b_coef):
    thr_a_row = thr_a.reshape(1, B)
    thr_e_row = thr_e.reshape(1, B)
    return pl.pallas_call(
        functools.partial(_adj_kernel, a_coef, b_coef),
        grid=(NB,),
        in_specs=[
            pl.BlockSpec((BM, B), lambda i: (i, 0)),
            pl.BlockSpec((1, B), lambda i: (0, 0)),
            pl.BlockSpec((BM, 1), lambda i: (i, 0)),
            pl.BlockSpec((1, B), lambda i: (0, 0)),
            pl.BlockSpec((BM, 1), lambda i: (i, 0)),
            pl.BlockSpec((1, B), lambda i: (0, 0)),
            pl.BlockSpec((BM, 1), lambda i: (i, 0)),
        ],
        out_specs=[
            pl.BlockSpec((BM, B), lambda i: (i, 0)),
            pl.BlockSpec((BM, 1), lambda i: (i, 0)),
        ],
        out_shape=[
            jax.ShapeDtypeStruct((B, B), jnp.float32),
            jax.ShapeDtypeStruct((B, 1), jnp.float32),
        ],
    )(s, lab_row, lab_col, thr_a_row, thr_a, thr_e_row, thr_e)


def _diff_kernel(ai_ref, aj_ref, dr_ref, di_ref, dj_ref, k_ref):
    dinv2 = dr_ref[...] * dr_ref[...]
    kk = _dot_nt(ai_ref[...] * dinv2, aj_ref[...])
    k_ref[...] = kk * di_ref[...] * dj_ref[...]


_NPAIR = NB * (NB + 1) // 2
_C = 2 * NB + 1


def _pair_i(t):
    i = t - t
    off = 0
    for m in range(1, NB):
        off += NB - m + 1
        i = i + jnp.where(t >= off, 1, 0)
    return i


def _pair_j(t):
    i = _pair_i(t)
    return t - (i * (_C - i)) // 2 + i


def _mirror_kernel(u_ref, ut_ref, k_ref):
    i = pl.program_id(0)
    j = pl.program_id(1)

    @pl.when(j >= i)
    def _copy():
        k_ref[...] = u_ref[...]

    @pl.when(j < i)
    def _transpose():
        k_ref[...] = ut_ref[...].T


def _diffusion(a, dinv_col):
    dinv_row = dinv_col.reshape(1, B)
    upper = pl.pallas_call(
        _diff_kernel,
        grid=(_NPAIR,),
        in_specs=[
            pl.BlockSpec((BM, B), lambda t: (_pair_i(t), 0)),
            pl.BlockSpec((BM, B), lambda t: (_pair_j(t), 0)),
            pl.BlockSpec((1, B), lambda t: (0, 0)),
            pl.BlockSpec((BM, 1), lambda t: (_pair_i(t), 0)),
            pl.BlockSpec((1, BM), lambda t: (0, _pair_j(t))),
        ],
        out_specs=pl.BlockSpec((BM, BM), lambda t: (_pair_i(t), _pair_j(t))),
        out_shape=jax.ShapeDtypeStruct((B, B), jnp.float32),
    )(a, a, dinv_row, dinv_col, dinv_row)
    return pl.pallas_call(
        _mirror_kernel,
        grid=(NB, NB),
        in_specs=[
            pl.BlockSpec((BM, BM), lambda i, j: (i, j)),
            pl.BlockSpec((BM, BM), lambda i, j: (j, i)),
        ],
        out_specs=pl.BlockSpec((BM, BM), lambda i, j: (i, j)),
        out_shape=jax.ShapeDtypeStruct((B, B), jnp.float32),
    )(upper, upper)


def _mm_nt_kernel(x_ref, w_ref, y_ref):
    y_ref[...] = _dot_nt(x_ref[...], w_ref[...])


def _mm_nt(x, w):
    n = w.shape[0]
    return pl.pallas_call(
        _mm_nt_kernel,
        grid=(NB,),
        in_specs=[
            pl.BlockSpec((BM, x.shape[1]), lambda i: (i, 0)),
            pl.BlockSpec(w.shape, lambda i: (0, 0)),
        ],
        out_specs=pl.BlockSpec((BM, n), lambda i: (i, 0)),
        out_shape=jax.ShapeDtypeStruct((B, n), jnp.float32),
    )(x, w)


def _applyA_res_kernel(a_ref, y_ref, dc_ref, di_ref, res_ref, z_ref):
    yd = y_ref[...] * dc_ref[...]
    z = jnp.dot(a_ref[...], yd, preferred_element_type=jnp.float32)
    z_ref[...] = z * di_ref[...] + res_ref[...]


def _applyA_kernel(a_ref, y_ref, dc_ref, di_ref, z_ref):
    yd = y_ref[...] * dc_ref[...]
    z = jnp.dot(a_ref[...], yd, preferred_element_type=jnp.float32)
    z_ref[...] = z * di_ref[...]


def _applyA(a, y, dinv_col, res=None):
    specs = [
        pl.BlockSpec((BM, B), lambda i: (i, 0)),
        pl.BlockSpec((B, DIM), lambda i: (0, 0)),
        pl.BlockSpec((B, 1), lambda i: (0, 0)),
        pl.BlockSpec((BM, 1), lambda i: (i, 0)),
    ]
    args = [a, y, dinv_col, dinv_col]
    fn = _applyA_kernel
    if res is not None:
        specs.append(pl.BlockSpec((BM, DIM), lambda i: (i, 0)))
        args.append(res)
        fn = _applyA_res_kernel
    return pl.pallas_call(
        fn,
        grid=(NB,),
        in_specs=specs,
        out_specs=pl.BlockSpec((BM, DIM), lambda i: (i, 0)),
        out_shape=jax.ShapeDtypeStruct((B, DIM), jnp.float32),
    )(*args)


def _bn_kernel(x_ref, g_ref, b_ref, o_ref):
    x = x_ref[...]
    mu = jnp.mean(x, axis=0, keepdims=True)
    xc = x - mu
    var = jnp.mean(xc * xc, axis=0, keepdims=True)
    bn = g_ref[...] * xc / jnp.sqrt(var + 1e-5) + b_ref[...]
    o_ref[...] = jnp.maximum(bn, 0.0)


def _bn_relu(x, g, b):
    return pl.pallas_call(
        _bn_kernel,
        grid=(1,),
        in_specs=[
            pl.BlockSpec((B, DIM), lambda i: (0, 0)),
            pl.BlockSpec((1, DIM), lambda i: (0, 0)),
            pl.BlockSpec((1, DIM), lambda i: (0, 0)),
        ],
        out_specs=pl.BlockSpec((B, DIM), lambda i: (0, 0)),
        out_shape=jax.ShapeDtypeStruct((B, DIM), jnp.float32),
    )(x, g.reshape(1, DIM), b.reshape(1, DIM))


def _cnt_kernel(labr_ref, labc_ref, c_ref):
    same = (labc_ref[...] == labr_ref[...]).astype(jnp.float32)
    c_ref[...] = jnp.sum(same, axis=1, keepdims=True)


def _counts(lab_row, lab_col):
    return pl.pallas_call(
        _cnt_kernel,
        grid=(NB,),
        in_specs=[
            pl.BlockSpec((1, B), lambda i: (0, 0)),
            pl.BlockSpec((BM, 1), lambda i: (i, 0)),
        ],
        out_specs=pl.BlockSpec((BM, 1), lambda i: (i, 0)),
        out_shape=jax.ShapeDtypeStruct((B, 1), jnp.float32),
    )(lab_row, lab_col)


def _idea_dinv(cnt):
    d = 0.98 * (cnt - 1.0) + 1.0 + 0.03 * (float(B) - cnt)
    return jax.lax.rsqrt(jnp.clip(d, 1e-8, None))


def _kloss_kernel(k0_ref, k1_ref, labr_ref, labc_ref, cr_ref, cc_ref,
                  p0_ref, p1_ref):
    i = pl.program_id(0)
    k0 = k0_ref[...]
    k1 = k1_ref[...]
    same = (labc_ref[...] == labr_ref[...]).astype(jnp.float32)
    col, row = _row_ids(i)
    eye = (col == row).astype(jnp.float32)
    v = jnp.maximum(same * 0.98 + (1.0 - same) * 0.03, eye)
    k_idea = v * _idea_dinv(cc_ref[...]) * _idea_dinv(cr_ref[...])
    d0 = k0 - k1
    d1 = k1 - k_idea
    p0_ref[...] = jnp.full((BM, 1), jnp.sum(d0 * d0), jnp.float32)
    p1_ref[...] = jnp.full((BM, 1), jnp.sum(d1 * d1), jnp.float32)


def _kloss(k0, k1, lab_row, lab_col, cnt):
    cnt_row = cnt.reshape(1, B)
    return pl.pallas_call(
        _kloss_kernel,
        grid=(NB,),
        in_specs=[
            pl.BlockSpec((BM, B), lambda i: (i, 0)),
            pl.BlockSpec((BM, B), lambda i: (i, 0)),
            pl.BlockSpec((1, B), lambda i: (0, 0)),
            pl.BlockSpec((BM, 1), lambda i: (i, 0)),
            pl.BlockSpec((1, B), lambda i: (0, 0)),
            pl.BlockSpec((BM, 1), lambda i: (i, 0)),
        ],
        out_specs=[
            pl.BlockSpec((BM, 1), lambda i: (i, 0)),
            pl.BlockSpec((BM, 1), lambda i: (i, 0)),
        ],
        out_shape=[
            jax.ShapeDtypeStruct((B, 1), jnp.float32),
            jax.ShapeDtypeStruct((B, 1), jnp.float32),
        ],
    )(k0, k1, lab_row, lab_col, cnt_row, cnt)


def _zloss_kernel(z0_ref, z1_ref, wp_ref, p_ref):
    wp = wp_ref[...]
    p0 = _dot_nt(z0_ref[...], wp)
    p1 = _dot_nt(z1_ref[...], wp)
    n0 = jnp.clip(jnp.sqrt(jnp.sum(p0 * p0, axis=1, keepdims=True)), 1e-12,
                  None)
    n1 = jnp.clip(jnp.sqrt(jnp.sum(p1 * p1, axis=1, keepdims=True)), 1e-12,
                  None)
    d = p0 / n0 - p1 / n1
    p_ref[...] = jnp.full((BM, 1), jnp.sum(d * d), jnp.float32)


def _zloss(z0, z1, wp):
    return pl.pallas_call(
        _zloss_kernel,
        grid=(NB,),
        in_specs=[
            pl.BlockSpec((BM, DIM), lambda i: (i, 0)),
            pl.BlockSpec((BM, DIM), lambda i: (i, 0)),
            pl.BlockSpec((PROJ, DIM), lambda i: (0, 0)),
        ],
        out_specs=pl.BlockSpec((BM, 1), lambda i: (i, 0)),
        out_shape=jax.ShapeDtypeStruct((B, 1), jnp.float32),
    )(z0, z1, wp)


def _psum(p):
    return jnp.sum(p[::BM, 0])


def _layer(x, lab_row, lab_col, w1, w2, g, b, a_coef, b_coef):
    s = _similarity(x)
    thr_a, thr_e = _thresholds(s, lab_row, lab_col)
    a, dinv = _adjacency(s, lab_row, lab_col, thr_a, thr_e, a_coef, b_coef)
    k = _diffusion(a, dinv)
    y1 = _mm_nt(x, w1)
    z1 = _applyA(a, y1, dinv)
    z1 = _bn_relu(z1, g, b)
    y2 = _mm_nt(z1, w2)
    z = _applyA(a, y2, dinv, res=x)
    return k, z


def kernel(feats_512_list, labels, W1_0, W2_0, g_0, b_0, W1_1, W2_1, g_1,
           b_1, Wp):
    labf = labels.astype(jnp.float32)
    lab_row = labf.reshape(1, B)
    lab_col = labf.reshape(B, 1)
    k0, z0 = _layer(feats_512_list[0], lab_row, lab_col, W1_0, W2_0, g_0,
                    b_0, 1.0, 1.0)
    k1, z1 = _layer(feats_512_list[1], lab_row, lab_col, W1_1, W2_1, g_1,
                    b_1, 1.2, 0.8)
    cnt = _counts(lab_row, lab_col)
    pk0, pk1 = _kloss(k0, k1, lab_row, lab_col, cnt)
    loss_align_K = _psum(pk0) / (float(B) * float(B))
    loss_idea = _psum(pk1) / (float(B) * float(B))
    pz = _zloss(z0, z1, Wp)
    loss_align_Z = _psum(pz) / (float(B) * float(PROJ))
    loss_pga = 0.5 * loss_align_K + 1.0 * loss_align_Z + 1.0 * loss_idea
    return (z0, z1, loss_align_K, loss_align_Z, loss_idea, loss_pga)


# speedup vs baseline: 1.1247x; 1.0283x over previous
import functools

import jax
import jax.numpy as jnp
from jax.experimental import pallas as pl

B = 2048
DIM = 512
PROJ = 768
TOPK = 64
BM = 256
NB = B // BM
FILL = 1.0e9
STEP = 1.0e4
VAL_ITERS = 22
IDX_ITERS = 12


def _dot_nt(a, b):
    return jax.lax.dot_general(a, b, (((1,), (1,)), ((), ())),
                               preferred_element_type=jnp.float32)


def _row_ids(i):
    col = jax.lax.broadcasted_iota(jnp.int32, (BM, B), 1).astype(jnp.float32)
    row = (jax.lax.broadcasted_iota(jnp.int32, (BM, B), 0) + i * BM).astype(
        jnp.float32)
    return col, row


def _sim_kernel(xi_ref, xj_ref, s_ref):
    xi = xi_ref[...]
    xj = xj_ref[...]
    ni = jnp.clip(jnp.sqrt(jnp.sum(xi * xi, axis=1, keepdims=True)), 1e-12,
                  None)
    nj = jnp.clip(jnp.sqrt(jnp.sum(xj * xj, axis=1, keepdims=True)), 1e-12,
                  None)
    s = _dot_nt(xi / ni, xj / nj)
    s_ref[...] = jnp.clip(s, -1.0 + 1e-8, 1.0 - 1e-8)


def _similarity(x):
    return pl.pallas_call(
        _sim_kernel,
        grid=(NB, NB),
        in_specs=[
            pl.BlockSpec((BM, DIM), lambda i, j: (i, 0)),
            pl.BlockSpec((BM, DIM), lambda i, j: (j, 0)),
        ],
        out_specs=pl.BlockSpec((BM, BM), lambda i, j: (i, j)),
        out_shape=jax.ShapeDtypeStruct((B, B), jnp.float32),
    )(x, x)


def _masked_pair(s, base, col):
    return base * s - (1.0 - base) * (FILL + STEP * col)


def _kth_threshold(masked):
    k = float(TOPK)
    n_real = jnp.sum((masked >= -2.0).astype(jnp.float32), axis=1,
                     keepdims=True)

    def vbody(_, c):
        lo, hi = c
        mid = (lo + hi) * 0.5
        cnt = jnp.sum((masked >= mid).astype(jnp.float32), axis=1,
                      keepdims=True)
        ok = cnt >= k
        return jnp.where(ok, mid, lo), jnp.where(ok, hi, mid)

    lo0 = jnp.full((BM, 1), -2.0, jnp.float32)
    hi0 = jnp.full((BM, 1), 2.0, jnp.float32)
    thr_val, _ = jax.lax.fori_loop(0, VAL_ITERS, vbody, (lo0, hi0))

    def ibody(_, c):
        lo, hi = c
        mid = jnp.floor((lo + hi) * 0.5)
        cnt = jnp.sum((masked >= -(FILL + STEP * mid)).astype(jnp.float32),
                      axis=1, keepdims=True)
        ok = cnt >= k
        return jnp.where(ok, lo, mid), jnp.where(ok, mid, hi)

    ilo0 = jnp.full((BM, 1), -1.0, jnp.float32)
    ihi0 = jnp.full((BM, 1), float(B), jnp.float32)
    _, t_star = jax.lax.fori_loop(0, IDX_ITERS, ibody, (ilo0, ihi0))
    thr_idx = -(FILL + STEP * t_star)
    return jnp.where(n_real >= k, thr_val, thr_idx)


def _thr_kernel(s_ref, labr_ref, labc_ref, ta_ref, te_ref):
    i = pl.program_id(0)
    s = s_ref[...]
    same = (labc_ref[...] == labr_ref[...]).astype(jnp.float32)
    col, row = _row_ids(i)
    noneye = (col != row).astype(jnp.float32)
    ta_ref[...] = _kth_threshold(_masked_pair(s, same * noneye, col))
    te_ref[...] = _kth_threshold(_masked_pair(s, (1.0 - same) * noneye, col))


def _thresholds(s, lab_row, lab_col):
    return pl.pallas_call(
        _thr_kernel,
        grid=(NB,),
        in_specs=[
            pl.BlockSpec((BM, B), lambda i: (i, 0)),
            pl.BlockSpec((1, B), lambda i: (0, 0)),
            pl.BlockSpec((BM, 1), lambda i: (i, 0)),
        ],
        out_specs=[
            pl.BlockSpec((BM, 1), lambda i: (i, 0)),
            pl.BlockSpec((BM, 1), lambda i: (i, 0)),
        ],
        out_shape=[
            jax.ShapeDtypeStruct((B, 1), jnp.float32),
            jax.ShapeDtypeStruct((B, 1), jnp.float32),
        ],
    )(s, lab_row, lab_col)


def _adj_kernel(a_coef, b_coef, s_ref, labr_ref, labc_ref, tar_ref, tac_ref,
                ter_ref, tec_ref, a_ref, dinv_ref):
    i = pl.program_id(0)
    s = s_ref[...]
    same = (labc_ref[...] == labr_ref[...]).astype(jnp.float32)
    col, row = _row_ids(i)
    noneye = (col != row).astype(jnp.float32)
    sa = same * noneye
    se = (1.0 - same) * noneye

    ma = _masked_pair(s, sa, col) >= tac_ref[...]
    ma_t = _masked_pair(s, sa, row) >= tar_ref[...]
    me = _masked_pair(s, se, col) >= tec_ref[...]
    me_t = _masked_pair(s, se, row) >= ter_ref[...]
    m_intra = jnp.logical_or(ma, ma_t).astype(jnp.float32)
    m_inter = jnp.logical_or(me, me_t).astype(jnp.float32)

    s_pos = jnp.maximum(s, 0.0)
    eye = 1.0 - noneye
    a = a_coef * s_pos * m_intra + b_coef * s_pos * m_inter + 1e-6 * eye
    a_ref[...] = a
    d = jnp.clip(jnp.sum(a, axis=1, keepdims=True), 1e-8, None)
    dinv_ref[...] = jax.lax.rsqrt(d)


def _adjacency(s, lab_row, lab_col, thr_a, thr_e, a_coef, b_coef):
    thr_a_row = thr_a.reshape(1, B)
    thr_e_row = thr_e.reshape(1, B)
    return pl.pallas_call(
        functools.partial(_adj_kernel, a_coef, b_coef),
        grid=(NB,),
        in_specs=[
            pl.BlockSpec((BM, B), lambda i: (i, 0)),
            pl.BlockSpec((1, B), lambda i: (0, 0)),
            pl.BlockSpec((BM, 1), lambda i: (i, 0)),
            pl.BlockSpec((1, B), lambda i: (0, 0)),
            pl.BlockSpec((BM, 1), lambda i: (i, 0)),
            pl.BlockSpec((1, B), lambda i: (0, 0)),
            pl.BlockSpec((BM, 1), lambda i: (i, 0)),
        ],
        out_specs=[
            pl.BlockSpec((BM, B), lambda i: (i, 0)),
            pl.BlockSpec((BM, 1), lambda i: (i, 0)),
        ],
        out_shape=[
            jax.ShapeDtypeStruct((B, B), jnp.float32),
            jax.ShapeDtypeStruct((B, 1), jnp.float32),
        ],
    )(s, lab_row, lab_col, thr_a_row, thr_a, thr_e_row, thr_e)


def _diff_kernel(ai_ref, aj_ref, dr_ref, di_ref, dj_ref, k_ref):
    dinv2 = dr_ref[...] * dr_ref[...]
    kk = _dot_nt(ai_ref[...] * dinv2, aj_ref[...])
    k_ref[...] = kk * di_ref[...] * dj_ref[...]


def _diffusion(a, dinv_col):
    dinv_row = dinv_col.reshape(1, B)
    return pl.pallas_call(
        _diff_kernel,
        grid=(NB, NB),
        in_specs=[
            pl.BlockSpec((BM, B), lambda i, j: (i, 0)),
            pl.BlockSpec((BM, B), lambda i, j: (j, 0)),
            pl.BlockSpec((1, B), lambda i, j: (0, 0)),
            pl.BlockSpec((BM, 1), lambda i, j: (i, 0)),
            pl.BlockSpec((1, BM), lambda i, j: (0, j)),
        ],
        out_specs=pl.BlockSpec((BM, BM), lambda i, j: (i, j)),
        out_shape=jax.ShapeDtypeStruct((B, B), jnp.float32),
    )(a, a, dinv_row, dinv_col, dinv_row)


def _mm_nt_kernel(x_ref, w_ref, y_ref):
    y_ref[...] = _dot_nt(x_ref[...], w_ref[...])


def _mm_nt(x, w):
    n = w.shape[0]
    return pl.pallas_call(
        _mm_nt_kernel,
        grid=(NB,),
        in_specs=[
            pl.BlockSpec((BM, x.shape[1]), lambda i: (i, 0)),
            pl.BlockSpec(w.shape, lambda i: (0, 0)),
        ],
        out_specs=pl.BlockSpec((BM, n), lambda i: (i, 0)),
        out_shape=jax.ShapeDtypeStruct((B, n), jnp.float32),
    )(x, w)


def _applyA_res_kernel(a_ref, y_ref, dc_ref, di_ref, res_ref, z_ref):
    yd = y_ref[...] * dc_ref[...]
    z = jnp.dot(a_ref[...], yd, preferred_element_type=jnp.float32)
    z_ref[...] = z * di_ref[...] + res_ref[...]


def _applyA_kernel(a_ref, y_ref, dc_ref, di_ref, z_ref):
    yd = y_ref[...] * dc_ref[...]
    z = jnp.dot(a_ref[...], yd, preferred_element_type=jnp.float32)
    z_ref[...] = z * di_ref[...]


def _applyA(a, y, dinv_col, res=None):
    specs = [
        pl.BlockSpec((BM, B), lambda i: (i, 0)),
        pl.BlockSpec((B, DIM), lambda i: (0, 0)),
        pl.BlockSpec((B, 1), lambda i: (0, 0)),
        pl.BlockSpec((BM, 1), lambda i: (i, 0)),
    ]
    args = [a, y, dinv_col, dinv_col]
    fn = _applyA_kernel
    if res is not None:
        specs.append(pl.BlockSpec((BM, DIM), lambda i: (i, 0)))
        args.append(res)
        fn = _applyA_res_kernel
    return pl.pallas_call(
        fn,
        grid=(NB,),
        in_specs=specs,
        out_specs=pl.BlockSpec((BM, DIM), lambda i: (i, 0)),
        out_shape=jax.ShapeDtypeStruct((B, DIM), jnp.float32),
    )(*args)


def _bn_kernel(x_ref, g_ref, b_ref, o_ref):
    x = x_ref[...]
    mu = jnp.mean(x, axis=0, keepdims=True)
    xc = x - mu
    var = jnp.mean(xc * xc, axis=0, keepdims=True)
    bn = g_ref[...] * xc / jnp.sqrt(var + 1e-5) + b_ref[...]
    o_ref[...] = jnp.maximum(bn, 0.0)


def _bn_relu(x, g, b):
    return pl.pallas_call(
        _bn_kernel,
        grid=(1,),
        in_specs=[
            pl.BlockSpec((B, DIM), lambda i: (0, 0)),
            pl.BlockSpec((1, DIM), lambda i: (0, 0)),
            pl.BlockSpec((1, DIM), lambda i: (0, 0)),
        ],
        out_specs=pl.BlockSpec((B, DIM), lambda i: (0, 0)),
        out_shape=jax.ShapeDtypeStruct((B, DIM), jnp.float32),
    )(x, g.reshape(1, DIM), b.reshape(1, DIM))


def _cnt_kernel(labr_ref, labc_ref, c_ref):
    same = (labc_ref[...] == labr_ref[...]).astype(jnp.float32)
    c_ref[...] = jnp.sum(same, axis=1, keepdims=True)


def _counts(lab_row, lab_col):
    return pl.pallas_call(
        _cnt_kernel,
        grid=(NB,),
        in_specs=[
            pl.BlockSpec((1, B), lambda i: (0, 0)),
            pl.BlockSpec((BM, 1), lambda i: (i, 0)),
        ],
        out_specs=pl.BlockSpec((BM, 1), lambda i: (i, 0)),
        out_shape=jax.ShapeDtypeStruct((B, 1), jnp.float32),
    )(lab_row, lab_col)


def _idea_dinv(cnt):
    d = 0.98 * (cnt - 1.0) + 1.0 + 0.03 * (float(B) - cnt)
    return jax.lax.rsqrt(jnp.clip(d, 1e-8, None))


def _kloss_kernel(k0_ref, k1_ref, labr_ref, labc_ref, cr_ref, cc_ref,
                  p0_ref, p1_ref):
    i = pl.program_id(0)
    k0 = k0_ref[...]
    k1 = k1_ref[...]
    same = (labc_ref[...] == labr_ref[...]).astype(jnp.float32)
    col, row = _row_ids(i)
    eye = (col == row).astype(jnp.float32)
    v = jnp.maximum(same * 0.98 + (1.0 - same) * 0.03, eye)
    k_idea = v * _idea_dinv(cc_ref[...]) * _idea_dinv(cr_ref[...])
    d0 = k0 - k1
    d1 = k1 - k_idea
    p0_ref[...] = jnp.full((BM, 1), jnp.sum(d0 * d0), jnp.float32)
    p1_ref[...] = jnp.full((BM, 1), jnp.sum(d1 * d1), jnp.float32)


def _kloss(k0, k1, lab_row, lab_col, cnt):
    cnt_row = cnt.reshape(1, B)
    return pl.pallas_call(
        _kloss_kernel,
        grid=(NB,),
        in_specs=[
            pl.BlockSpec((BM, B), lambda i: (i, 0)),
            pl.BlockSpec((BM, B), lambda i: (i, 0)),
            pl.BlockSpec((1, B), lambda i: (0, 0)),
            pl.BlockSpec((BM, 1), lambda i: (i, 0)),
            pl.BlockSpec((1, B), lambda i: (0, 0)),
            pl.BlockSpec((BM, 1), lambda i: (i, 0)),
        ],
        out_specs=[
            pl.BlockSpec((BM, 1), lambda i: (i, 0)),
            pl.BlockSpec((BM, 1), lambda i: (i, 0)),
        ],
        out_shape=[
            jax.ShapeDtypeStruct((B, 1), jnp.float32),
            jax.ShapeDtypeStruct((B, 1), jnp.float32),
        ],
    )(k0, k1, lab_row, lab_col, cnt_row, cnt)


def _zloss_kernel(z0_ref, z1_ref, wp_ref, p_ref):
    wp = wp_ref[...]
    p0 = _dot_nt(z0_ref[...], wp)
    p1 = _dot_nt(z1_ref[...], wp)
    n0 = jnp.clip(jnp.sqrt(jnp.sum(p0 * p0, axis=1, keepdims=True)), 1e-12,
                  None)
    n1 = jnp.clip(jnp.sqrt(jnp.sum(p1 * p1, axis=1, keepdims=True)), 1e-12,
                  None)
    d = p0 / n0 - p1 / n1
    p_ref[...] = jnp.full((BM, 1), jnp.sum(d * d), jnp.float32)


def _zloss(z0, z1, wp):
    return pl.pallas_call(
        _zloss_kernel,
        grid=(NB,),
        in_specs=[
            pl.BlockSpec((BM, DIM), lambda i: (i, 0)),
            pl.BlockSpec((BM, DIM), lambda i: (i, 0)),
            pl.BlockSpec((PROJ, DIM), lambda i: (0, 0)),
        ],
        out_specs=pl.BlockSpec((BM, 1), lambda i: (i, 0)),
        out_shape=jax.ShapeDtypeStruct((B, 1), jnp.float32),
    )(z0, z1, wp)


def _psum(p):
    return jnp.sum(p[::BM, 0])


def _layer(x, lab_row, lab_col, w1, w2, g, b, a_coef, b_coef):
    s = _similarity(x)
    thr_a, thr_e = _thresholds(s, lab_row, lab_col)
    a, dinv = _adjacency(s, lab_row, lab_col, thr_a, thr_e, a_coef, b_coef)
    k = _diffusion(a, dinv)
    y1 = _mm_nt(x, w1)
    z1 = _applyA(a, y1, dinv)
    z1 = _bn_relu(z1, g, b)
    y2 = _mm_nt(z1, w2)
    z = _applyA(a, y2, dinv, res=x)
    return k, z


def kernel(feats_512_list, labels, W1_0, W2_0, g_0, b_0, W1_1, W2_1, g_1,
           b_1, Wp):
    labf = labels.astype(jnp.float32)
    lab_row = labf.reshape(1, B)
    lab_col = labf.reshape(B, 1)
    k0, z0 = _layer(feats_512_list[0], lab_row, lab_col, W1_0, W2_0, g_0,
                    b_0, 1.0, 1.0)
    k1, z1 = _layer(feats_512_list[1], lab_row, lab_col, W1_1, W2_1, g_1,
                    b_1, 1.2, 0.8)
    cnt = _counts(lab_row, lab_col)
    pk0, pk1 = _kloss(k0, k1, lab_row, lab_col, cnt)
    loss_align_K = _psum(pk0) / (float(B) * float(B))
    loss_idea = _psum(pk1) / (float(B) * float(B))
    pz = _zloss(z0, z1, Wp)
    loss_align_Z = _psum(pz) / (float(B) * float(PROJ))
    loss_pga = 0.5 * loss_align_K + 1.0 * loss_align_Z + 1.0 * loss_idea
    return (z0, z1, loss_align_K, loss_align_Z, loss_idea, loss_pga)
